# baseline (speedup 1.0000x reference)
import functools

import jax
import jax.numpy as jnp
from jax.experimental import pallas as pl
from jax.experimental.pallas import tpu as pltpu

_EPS_LN = 1e-5
_EPS_BN = 1e-5
_VC = 8192


def _layer_norm_t(v, w, b):
    mu = jnp.mean(v, axis=0, keepdims=True)
    var = jnp.mean((v - mu) ** 2, axis=0, keepdims=True)
    return (v - mu) / jnp.sqrt(var + _EPS_LN) * w + b


def _encoder_kernel(nv, V, B, P, H,
                    x_ref, tbl_ref,
                    fc1w_ref, fc1b_ref, fc2w_ref, fc2b_ref,
                    ln1w_ref, ln1b_ref, ln2w_ref, ln2b_ref,
                    bn1w_ref, bn1b_ref, bn2w_ref, bn2b_ref,
                    out_ref, acc_ref):
    iv = pl.program_id(0)
    p = pl.program_id(1)

    gv = iv * _VC + jax.lax.broadcasted_iota(jnp.int32, (1, _VC), 1)
    tkeep = jnp.logical_and(gv >= 1, gv < V)
    vkeep = gv < V

    t = tbl_ref[...]
    tb = jnp.where(tkeep, t, 0.0).astype(jnp.bfloat16)
    ones = jnp.where(vkeep, 1.0, 0.0).astype(jnp.bfloat16)
    taug = jnp.concatenate([tb, ones], axis=0)

    mb = x_ref[0].astype(jnp.bfloat16)

    contrib = jnp.dot(taug, mb, preferred_element_type=jnp.float32)

    @pl.when(iv == 0)
    def _():
        acc_ref[p] = contrib

    @pl.when(iv > 0)
    def _():
        acc_ref[p] = acc_ref[p] + contrib

    @pl.when(jnp.logical_and(iv == nv - 1, p == P - 1))
    def _():
        acc = acc_ref[...]
        sums = acc[:, :H, :]
        cnt = acc[:, H:H + 1, :]
        pe = jnp.where(cnt > 0.0, sums / jnp.maximum(cnt, 1.0), 0.0)
        xm = pe[0]
        for q in range(1, P):
            xm = xm + pe[q]
        xm = xm * (1.0 / P)

        h = _layer_norm_t(xm, ln1w_ref[...], ln1b_ref[...])
        h = jnp.dot(fc1w_ref[...], h, preferred_element_type=jnp.float32,
                    precision=jax.lax.Precision.HIGHEST) + fc1b_ref[...]
        h = jnp.maximum(h, 0.0)
        h = h * (bn1w_ref[...] / jnp.sqrt(1.0 + _EPS_BN)) + bn1b_ref[...]

        h = _layer_norm_t(h, ln2w_ref[...], ln2b_ref[...])
        h = jnp.dot(fc2w_ref[...], h, preferred_element_type=jnp.float32,
                    precision=jax.lax.Precision.HIGHEST) + fc2b_ref[...]
        h = jnp.maximum(h, 0.0)
        h = h * (bn2w_ref[...] / jnp.sqrt(1.0 + _EPS_BN)) + bn2b_ref[...]

        out_ref[...] = h


def kernel(inputs, entity_emb, fc1_w, fc1_b, fc2_w, fc2_b,
           ln1_w, ln1_b, ln2_w, ln2_b, bn1_w, bn1_b, bn2_w, bn2_b):
    B, P, V = inputs.shape
    H = entity_emb.shape[1]
    nv = pl.cdiv(V, _VC)

    xT = inputs.transpose(1, 2, 0)
    tT = entity_emb.T

    c = lambda a: a.reshape(H, 1)
    full = lambda shape: pl.BlockSpec(shape, lambda iv, p: (0, 0))

    outT = pl.pallas_call(
        functools.partial(_encoder_kernel, nv, V, B, P, H),
        grid=(nv, P),
        in_specs=[
            pl.BlockSpec((1, _VC, B), lambda iv, p: (p, iv, 0)),
            pl.BlockSpec((H, _VC), lambda iv, p: (0, iv)),
            full((H, H)), full((H, 1)), full((H, H)), full((H, 1)),
            full((H, 1)), full((H, 1)), full((H, 1)), full((H, 1)),
            full((H, 1)), full((H, 1)), full((H, 1)), full((H, 1)),
        ],
        out_specs=pl.BlockSpec((H, B), lambda iv, p: (0, 0)),
        out_shape=jax.ShapeDtypeStruct((H, B), jnp.float32),
        scratch_shapes=[pltpu.VMEM((P, H + 1, B), jnp.float32)],
        compiler_params=pltpu.CompilerParams(
            dimension_semantics=("arbitrary", "arbitrary")),
    )(xT, tT,
      fc1_w, c(fc1_b), fc2_w, c(fc2_b),
      c(ln1_w), c(ln1_b), c(ln2_w), c(ln2_b),
      c(bn1_w), c(bn1_b), c(bn2_w), c(bn2_b))
    return outT.T


# speedup vs baseline: 1.1114x; 1.1114x over previous
import functools

import jax
import jax.numpy as jnp
from jax.experimental import pallas as pl
from jax.experimental.pallas import tpu as pltpu

_EPS_LN = 1e-5
_EPS_BN = 1e-5
_VC = 16384


def _layer_norm_t(v, w, b):
    mu = jnp.mean(v, axis=0, keepdims=True)
    var = jnp.mean((v - mu) ** 2, axis=0, keepdims=True)
    return (v - mu) / jnp.sqrt(var + _EPS_LN) * w + b


def _encoder_kernel(nv, V, B, P, H,
                    x_ref, tbl_ref,
                    fc1w_ref, fc1b_ref, fc2w_ref, fc2b_ref,
                    ln1w_ref, ln1b_ref, ln2w_ref, ln2b_ref,
                    bn1w_ref, bn1b_ref, bn2w_ref, bn2b_ref,
                    out_ref, acc_ref):
    iv = pl.program_id(0)
    p = pl.program_id(1)

    gv = iv * _VC + jax.lax.broadcasted_iota(jnp.int32, (1, _VC), 1)
    tkeep = jnp.logical_and(gv >= 1, gv < V)
    vkeep = gv < V

    t = tbl_ref[...]
    tb = jnp.where(tkeep, t, 0.0).astype(jnp.bfloat16)
    ones = jnp.where(vkeep, 1.0, 0.0).astype(jnp.bfloat16)
    taug = jnp.concatenate([tb, ones], axis=0)

    mb = x_ref[0].astype(jnp.bfloat16)

    contrib = jnp.dot(taug, mb, preferred_element_type=jnp.float32)

    @pl.when(iv == 0)
    def _():
        acc_ref[p] = contrib

    @pl.when(iv > 0)
    def _():
        acc_ref[p] = acc_ref[p] + contrib

    @pl.when(jnp.logical_and(iv == nv - 1, p == P - 1))
    def _():
        acc = acc_ref[...]
        sums = acc[:, :H, :]
        cnt = acc[:, H:H + 1, :]
        pe = jnp.where(cnt > 0.0, sums / jnp.maximum(cnt, 1.0), 0.0)
        xm = pe[0]
        for q in range(1, P):
            xm = xm + pe[q]
        xm = xm * (1.0 / P)

        h = _layer_norm_t(xm, ln1w_ref[...], ln1b_ref[...])
        h = jnp.dot(fc1w_ref[...], h, preferred_element_type=jnp.float32,
                    precision=jax.lax.Precision.HIGHEST) + fc1b_ref[...]
        h = jnp.maximum(h, 0.0)
        h = h * (bn1w_ref[...] / jnp.sqrt(1.0 + _EPS_BN)) + bn1b_ref[...]

        h = _layer_norm_t(h, ln2w_ref[...], ln2b_ref[...])
        h = jnp.dot(fc2w_ref[...], h, preferred_element_type=jnp.float32,
                    precision=jax.lax.Precision.HIGHEST) + fc2b_ref[...]
        h = jnp.maximum(h, 0.0)
        h = h * (bn2w_ref[...] / jnp.sqrt(1.0 + _EPS_BN)) + bn2b_ref[...]

        out_ref[...] = h


def kernel(inputs, entity_emb, fc1_w, fc1_b, fc2_w, fc2_b,
           ln1_w, ln1_b, ln2_w, ln2_b, bn1_w, bn1_b, bn2_w, bn2_b):
    B, P, V = inputs.shape
    H = entity_emb.shape[1]
    nv = pl.cdiv(V, _VC)

    xT = inputs.transpose(1, 2, 0)
    tT = entity_emb.T

    c = lambda a: a.reshape(H, 1)
    full = lambda shape: pl.BlockSpec(shape, lambda iv, p: (0, 0))

    outT = pl.pallas_call(
        functools.partial(_encoder_kernel, nv, V, B, P, H),
        grid=(nv, P),
        in_specs=[
            pl.BlockSpec((1, _VC, B), lambda iv, p: (p, iv, 0)),
            pl.BlockSpec((H, _VC), lambda iv, p: (0, iv)),
            full((H, H)), full((H, 1)), full((H, H)), full((H, 1)),
            full((H, 1)), full((H, 1)), full((H, 1)), full((H, 1)),
            full((H, 1)), full((H, 1)), full((H, 1)), full((H, 1)),
        ],
        out_specs=pl.BlockSpec((H, B), lambda iv, p: (0, 0)),
        out_shape=jax.ShapeDtypeStruct((H, B), jnp.float32),
        scratch_shapes=[pltpu.VMEM((P, H + 1, B), jnp.float32)],
        compiler_params=pltpu.CompilerParams(
            dimension_semantics=("arbitrary", "arbitrary")),
    )(xT, tT,
      fc1_w, c(fc1_b), fc2_w, c(fc2_b),
      c(ln1_w), c(ln1_b), c(ln2_w), c(ln2_b),
      c(bn1_w), c(bn1_b), c(bn2_w), c(bn2_b))
    return outT.T
